# Initial kernel scaffold; baseline (speedup 1.0000x reference)
import jax
import jax.numpy as jnp
from jax.experimental import pallas as pl


def kernel(aa_indices, mod_x, table):
    raise NotImplementedError("write your pallas kernel here")


# speedup vs baseline: 2.0972x; 2.0972x over previous
import functools

import jax
import jax.numpy as jnp
from jax import lax
from jax.experimental import pallas as pl
from jax.experimental.pallas import tpu as pltpu
from jax.experimental.pallas import tpu_sc as plsc

MOD_FEAT = 109
OUT_FEATURES = 256
HIDDEN = OUT_FEATURES - MOD_FEAT


def _make_sc_kernel(R, V, NC, NS, N):
    NW = NC * NS
    rows_per_w = R // NW
    steps = rows_per_w // N
    mesh = plsc.VectorSubcoreMesh(core_axis_name="c", subcore_axis_name="s")

    @functools.partial(
        pl.kernel,
        mesh=mesh,
        out_type=jax.ShapeDtypeStruct((R, OUT_FEATURES), jnp.float32),
        scratch_types=[
            pltpu.VMEM((N,), jnp.int32),
            pltpu.VMEM((N, OUT_FEATURES), jnp.float32),
            pltpu.VMEM((N, MOD_FEAT), jnp.float32),
            pltpu.VMEM_SHARED((V, OUT_FEATURES), jnp.float32),
            pltpu.SemaphoreType.DMA,
        ],
        compiler_params=pltpu.CompilerParams(use_tc_tiling_on_sc=False),
    )
    def sc_k(idx_hbm, mod_hbm, table_hbm, out_hbm,
             idx_v, out_v, mod_v, tab_sh, sem):
        cid = lax.axis_index("c")
        sid = lax.axis_index("s")
        wid = sid * NC + cid
        base0 = wid * rows_per_w

        @pl.when(sid == 0)
        def _():
            pltpu.sync_copy(table_hbm, tab_sh)
        plsc.subcore_barrier()

        def copy_row(r, carry):
            src = mod_v.at[r]
            dst = out_v.at[r]
            for k in range(6):
                dst[pl.ds(HIDDEN + 16 * k, 16)] = src[pl.ds(16 * k, 16)]
            dst[pl.ds(OUT_FEATURES - 16, 16)] = src[pl.ds(MOD_FEAT - 16, 16)]
            return carry

        def body(t, carry):
            base = base0 + t * N
            pltpu.sync_copy(idx_hbm.at[pl.ds(base, N)], idx_v)
            pltpu.async_copy(tab_sh.at[idx_v], out_v, sem).wait()
            pltpu.sync_copy(mod_hbm.at[pl.ds(base, N)], mod_v)
            lax.fori_loop(0, N, copy_row, 0)
            pltpu.sync_copy(out_v, out_hbm.at[pl.ds(base, N)])
            return carry

        lax.fori_loop(0, steps, body, 0)

    return sc_k


def kernel(aa_indices, mod_x, table):
    B, L = aa_indices.shape
    R = B * L
    idx = aa_indices.reshape(R).astype(jnp.int32)
    mod = mod_x.reshape(R, MOD_FEAT)
    table256 = jnp.zeros((table.shape[0], OUT_FEATURES), table.dtype)
    table256 = table256.at[1:, :HIDDEN].set(table[1:])

    info = plsc.get_sparse_core_info()
    sc_k = _make_sc_kernel(R, table.shape[0], info.num_cores,
                           info.num_subcores, 160)
    out = sc_k(idx, mod, table256)
    return out.reshape(B, L, OUT_FEATURES)


# speedup vs baseline: 2.4153x; 1.1517x over previous
import functools

import jax
import jax.numpy as jnp
from jax import lax
from jax.experimental import pallas as pl
from jax.experimental.pallas import tpu as pltpu
from jax.experimental.pallas import tpu_sc as plsc

MOD_FEAT = 109
OUT_FEATURES = 256
HIDDEN = OUT_FEATURES - MOD_FEAT


def _make_sc_kernel(R, V, NC, NS, N):
    NW = NC * NS
    rows_per_w = R // NW
    steps = rows_per_w // N
    assert steps % 2 == 0
    mesh = plsc.VectorSubcoreMesh(core_axis_name="c", subcore_axis_name="s")

    @functools.partial(
        pl.kernel,
        mesh=mesh,
        out_type=jax.ShapeDtypeStruct((R, OUT_FEATURES), jnp.float32),
        scratch_types=[
            pltpu.VMEM((N,), jnp.int32),
            pltpu.VMEM((N,), jnp.int32),
            pltpu.VMEM((N, OUT_FEATURES), jnp.float32),
            pltpu.VMEM((N, OUT_FEATURES), jnp.float32),
            pltpu.VMEM((N, MOD_FEAT), jnp.float32),
            pltpu.VMEM((N, MOD_FEAT), jnp.float32),
            pltpu.VMEM_SHARED((V, OUT_FEATURES), jnp.float32),
            pltpu.SemaphoreType.DMA,
            pltpu.SemaphoreType.DMA,
            pltpu.SemaphoreType.DMA,
            pltpu.SemaphoreType.DMA,
            pltpu.SemaphoreType.DMA,
            pltpu.SemaphoreType.DMA,
            pltpu.SemaphoreType.DMA,
            pltpu.SemaphoreType.DMA,
        ],
        compiler_params=pltpu.CompilerParams(use_tc_tiling_on_sc=False),
    )
    def sc_k(idx_hbm, mod_hbm, table_hbm, out_hbm,
             idx_v0, idx_v1, out_v0, out_v1, mod_v0, mod_v1, tab_sh,
             idx_s0, idx_s1, mod_s0, mod_s1, g_s0, g_s1, w_s0, w_s1):
        cid = lax.axis_index("c")
        sid = lax.axis_index("s")
        wid = sid * NC + cid
        base0 = wid * rows_per_w

        @pl.when(sid == 0)
        def _():
            pltpu.sync_copy(table_hbm, tab_sh)
        plsc.subcore_barrier()

        slot0 = (idx_v0, mod_v0, out_v0, idx_s0, mod_s0, g_s0, w_s0)
        slot1 = (idx_v1, mod_v1, out_v1, idx_s1, mod_s1, g_s1, w_s1)

        def copy_tile(mod_v, out_v):
            def cr(i, carry):
                for j in range(4):
                    r = i * 4 + j
                    src = mod_v.at[r]
                    dst = out_v.at[r]
                    for k in range(6):
                        dst[pl.ds(HIDDEN + 16 * k, 16)] = src[pl.ds(16 * k, 16)]
                    dst[pl.ds(OUT_FEATURES - 16, 16)] = src[pl.ds(MOD_FEAT - 16, 16)]
                return carry
            lax.fori_loop(0, N // 4, cr, 0)

        def step(t, cur, oth):
            idx_c, mod_c, out_c, idx_sc, mod_sc, g_sc, w_sc = cur
            idx_o, mod_o, out_o, idx_so, mod_so, g_so, w_so = oth
            base = base0 + t * N
            pltpu.make_async_copy(tab_sh.at[idx_c], out_c, g_sc).wait()
            pltpu.make_async_copy(mod_hbm.at[pl.ds(base, N)], mod_c, mod_sc).wait()
            @pl.when(t >= 1)
            def _():
                pltpu.make_async_copy(out_o, out_hbm.at[pl.ds(base, N)], w_so).wait()
            @pl.when(t < steps - 1)
            def _():
                pltpu.make_async_copy(idx_hbm.at[pl.ds(base, N)], idx_o, idx_so).wait()
                pltpu.async_copy(tab_sh.at[idx_o], out_o, g_so)
            copy_tile(mod_c, out_c)
            pltpu.async_copy(out_c, out_hbm.at[pl.ds(base, N)], w_sc)
            @pl.when(t < steps - 2)
            def _():
                pltpu.async_copy(idx_hbm.at[pl.ds(base + 2 * N, N)], idx_c, idx_sc)
                pltpu.async_copy(mod_hbm.at[pl.ds(base + 2 * N, N)], mod_c, mod_sc)

        pltpu.async_copy(idx_hbm.at[pl.ds(base0, N)], idx_v0, idx_s0)
        pltpu.async_copy(mod_hbm.at[pl.ds(base0, N)], mod_v0, mod_s0)
        pltpu.async_copy(idx_hbm.at[pl.ds(base0 + N, N)], idx_v1, idx_s1)
        pltpu.async_copy(mod_hbm.at[pl.ds(base0 + N, N)], mod_v1, mod_s1)
        pltpu.make_async_copy(idx_hbm.at[pl.ds(base0, N)], idx_v0, idx_s0).wait()
        pltpu.async_copy(tab_sh.at[idx_v0], out_v0, g_s0)

        def pair(p, carry):
            step(2 * p, slot0, slot1)
            step(2 * p + 1, slot1, slot0)
            return carry

        lax.fori_loop(0, steps // 2, pair, 0)
        pltpu.make_async_copy(out_v1, out_hbm.at[pl.ds(base0, N)], w_s1).wait()

    return sc_k


def kernel(aa_indices, mod_x, table):
    B, L = aa_indices.shape
    R = B * L
    idx = aa_indices.reshape(R).astype(jnp.int32)
    mod = mod_x.reshape(R, MOD_FEAT)
    table256 = jnp.zeros((table.shape[0], OUT_FEATURES), table.dtype)
    table256 = table256.at[1:, :HIDDEN].set(table[1:])

    info = plsc.get_sparse_core_info()
    sc_k = _make_sc_kernel(R, table.shape[0], info.num_cores,
                           info.num_subcores, 160)
    out = sc_k(idx, mod, table256)
    return out.reshape(B, L, OUT_FEATURES)


# speedup vs baseline: 3.6786x; 1.5230x over previous
import functools

import jax
import jax.numpy as jnp
from jax import lax
from jax.experimental import pallas as pl
from jax.experimental.pallas import tpu as pltpu
from jax.experimental.pallas import tpu_sc as plsc

MOD_FEAT = 109
OUT_FEATURES = 256
HIDDEN = OUT_FEATURES - MOD_FEAT
N = 64
NK = 7
NO = 4
NI = 2


def _make_sc_kernel(B, L, V, NC, NS):
    NW = NC * NS
    b_per_w = B // NW
    CH = b_per_w // N
    assert CH % NO == 0 and CH % NI == 0
    mesh = plsc.VectorSubcoreMesh(core_axis_name="c", subcore_axis_name="s")

    @functools.partial(
        pl.kernel,
        mesh=mesh,
        out_type=jax.ShapeDtypeStruct((L, B, OUT_FEATURES), jnp.float32),
        scratch_types=(
            [pltpu.VMEM((N,), jnp.int32) for _ in range(NI)]
            + [pltpu.VMEM((N, OUT_FEATURES), jnp.float32) for _ in range(NO)]
            + [pltpu.VMEM((MOD_FEAT, N + 1), jnp.float32) for _ in range(NI)]
            + [pltpu.VMEM((NK, 16), jnp.int32),
               pltpu.VMEM_SHARED((V, OUT_FEATURES), jnp.float32)]
            + [pltpu.SemaphoreType.DMA for _ in range(2 * NI + 2 * NO)]
        ),
        compiler_params=pltpu.CompilerParams(use_tc_tiling_on_sc=False,
                                             needs_layout_passes=False),
    )
    def sc_k(idx_hbm, mod_hbm, table_hbm, out_hbm,
             idx_v0, idx_v1, out_v0, out_v1, out_v2, out_v3,
             mod_v0, mod_v1, ridx_v, tab_sh,
             i_s0, i_s1, m_s0, m_s1, g_s0, g_s1, g_s2, g_s3,
             w_s0, w_s1, w_s2, w_s3):
        cid = lax.axis_index("c")
        sid = lax.axis_index("s")
        wid = sid * NC + cid
        bw0 = wid * b_per_w

        idx_v = [idx_v0, idx_v1]
        mod_v = [mod_v0, mod_v1]
        out_v = [out_v0, out_v1, out_v2, out_v3]
        i_s = [i_s0, i_s1]
        m_s = [m_s0, m_s1]
        g_s = [g_s0, g_s1, g_s2, g_s3]
        w_s = [w_s0, w_s1, w_s2, w_s3]

        @pl.when(sid == 0)
        def _():
            pltpu.sync_copy(table_hbm, tab_sh)
        iota = lax.iota(jnp.int32, 16)
        for k in range(NK):
            ridx_v[k] = iota + min(16 * k, MOD_FEAT - 16)
        plsc.subcore_barrier()

        def transpose_tile(mod_t, out_t):
            rows = tuple(ridx_v[k] for k in range(NK))

            def per_b(b, carry):
                for u in range(2):
                    bb = b * 2 + u
                    cols = jnp.full((16,), bb, jnp.int32)
                    dst = out_t.at[bb]
                    for k in range(NK):
                        vals = plsc.load_gather(mod_t, [carry[k], cols])
                        dst[pl.ds(HIDDEN + min(16 * k, MOD_FEAT - 16), 16)] = vals
                return carry
            lax.fori_loop(0, N // 2, per_b, rows)

        def addr(l, c, delta):
            c2 = c + delta
            return l + c2 // CH, bw0 + (c2 % CH) * N

        def step(l, c):
            so = c % NO
            si = c % NI
            so1 = (c + 1) % NO
            si1 = (c + 1) % NI
            b0 = bw0 + c * N
            pltpu.make_async_copy(tab_sh.at[idx_v[si]], out_v[so], g_s[so]).wait()
            pltpu.make_async_copy(
                mod_hbm.at[l, :, pl.ds(b0, N)],
                mod_v[si].at[:, pl.ds(0, N)], m_s[si]).wait()

            def wait_write():
                pltpu.make_async_copy(
                    out_v[so1], out_hbm.at[l, pl.ds(b0, N), :], w_s[so1]).wait()
            if c >= 3:
                wait_write()
            else:
                pl.when(l >= 1)(wait_write)

            l1, b1 = addr(l, c, 1)

            def issue_gather():
                pltpu.make_async_copy(
                    idx_hbm.at[l1, pl.ds(b1, N)], idx_v[si1], i_s[si1]).wait()
                pltpu.async_copy(tab_sh.at[idx_v[si1]], out_v[so1], g_s[so1])
            if c == CH - 1:
                pl.when(l <= L - 2)(issue_gather)
            else:
                issue_gather()

            transpose_tile(mod_v[si], out_v[so])
            pltpu.async_copy(out_v[so], out_hbm.at[l, pl.ds(b0, N), :], w_s[so])

            l2, b2 = addr(l, c, 2)

            def prefetch():
                pltpu.async_copy(idx_hbm.at[l2, pl.ds(b2, N)], idx_v[si], i_s[si])
                pltpu.async_copy(
                    mod_hbm.at[l2, :, pl.ds(b2, N)],
                    mod_v[si].at[:, pl.ds(0, N)], m_s[si])
            if c >= CH - 2:
                pl.when(l <= L - 2)(prefetch)
            else:
                prefetch()

        pltpu.async_copy(idx_hbm.at[0, pl.ds(bw0, N)], idx_v0, i_s0)
        pltpu.async_copy(mod_hbm.at[0, :, pl.ds(bw0, N)],
                         mod_v0.at[:, pl.ds(0, N)], m_s0)
        pltpu.async_copy(idx_hbm.at[0, pl.ds(bw0 + N, N)], idx_v1, i_s1)
        pltpu.async_copy(mod_hbm.at[0, :, pl.ds(bw0 + N, N)],
                         mod_v1.at[:, pl.ds(0, N)], m_s1)
        pltpu.make_async_copy(idx_hbm.at[0, pl.ds(bw0, N)], idx_v0, i_s0).wait()
        pltpu.async_copy(tab_sh.at[idx_v0], out_v0, g_s0)

        def plane(l, carry):
            for c in range(CH):
                step(l, c)
            return carry

        lax.fori_loop(0, L, plane, 0)
        for c in range(CH - NO + 1, CH):
            pltpu.make_async_copy(
                out_v[c % NO],
                out_hbm.at[L - 1, pl.ds(bw0 + c * N, N), :], w_s[c % NO]).wait()

    return sc_k


def kernel(aa_indices, mod_x, table):
    B, L = aa_indices.shape
    idx_t = jnp.transpose(aa_indices.astype(jnp.int32), (1, 0))
    mod_t = jnp.transpose(mod_x, (1, 2, 0))
    table256 = jnp.zeros((table.shape[0], OUT_FEATURES), table.dtype)
    table256 = table256.at[1:, :HIDDEN].set(table[1:])

    info = plsc.get_sparse_core_info()
    sc_k = _make_sc_kernel(B, L, table.shape[0], info.num_cores,
                           info.num_subcores)
    out = sc_k(idx_t, mod_t, table256)
    return jnp.transpose(out, (1, 0, 2))
